# baseline (speedup 1.0000x reference)
import functools

import jax
import jax.numpy as jnp
from jax import lax
from jax.experimental import pallas as pl
from jax.experimental.pallas import tpu as pltpu
from jax.experimental.pallas import tpu_sc as plsc

_STATES = 1000000
_ACTIONS = 16
_BATCH = 16384


@functools.cache
def _build_gather():
    info = plsc.get_sparse_core_info()
    num_cores, num_subcores = info.num_cores, info.num_subcores
    num_workers = num_cores * num_subcores
    b_per_w = _BATCH // num_workers
    n_blk = b_per_w // 16
    n_elems = b_per_w * _ACTIONS
    mesh = plsc.VectorSubcoreMesh(core_axis_name="c", subcore_axis_name="s")

    @functools.partial(
        pl.kernel,
        mesh=mesh,
        out_type=jax.ShapeDtypeStruct((_BATCH * _ACTIONS,), jnp.float32),
        compiler_params=pltpu.CompilerParams(needs_layout_passes=False),
        scratch_types=[
            pltpu.VMEM((b_per_w,), jnp.int32),
            pltpu.VMEM((n_elems,), jnp.int32),
            pltpu.VMEM((n_elems,), jnp.float32),
            pltpu.SemaphoreType.DMA,
        ],
    )
    def gather_kernel(values_hbm, idx_hbm, out_hbm, idx_v, fidx_v, got_v, sem):
        wid = lax.axis_index("s") * num_cores + lax.axis_index("c")
        base = wid * b_per_w
        pltpu.sync_copy(idx_hbm.at[pl.ds(base, b_per_w)], idx_v)

        def fidx_body(k, _):
            v = idx_v[pl.ds(k * 16, 16)]
            for j in range(_ACTIONS):
                fidx_v[pl.ds(j * b_per_w + k * 16, 16)] = v + j * _STATES
            return _

        lax.fori_loop(0, n_blk, fidx_body, None)

        pltpu.async_copy(values_hbm.at[fidx_v], got_v, sem).wait()

        for j in range(_ACTIONS):
            pltpu.sync_copy(
                got_v.at[pl.ds(j * b_per_w, b_per_w)],
                out_hbm.at[pl.ds(j * _BATCH + base, b_per_w)],
            )

    return gather_kernel


def kernel(state, values):
    idx = state.astype(jnp.int32)
    flat = values.T.reshape(_STATES * _ACTIONS)
    out = _build_gather()(flat, idx)
    return out.reshape(_ACTIONS, _BATCH).T


# speedup vs baseline: 2.7357x; 2.7357x over previous
import functools

import jax
import jax.numpy as jnp
from jax import lax
from jax.experimental import pallas as pl
from jax.experimental.pallas import tpu as pltpu
from jax.experimental.pallas import tpu_sc as plsc

_STATES = 1000000
_ACTIONS = 16
_BATCH = 16384


@functools.cache
def _build_gather():
    info = plsc.get_sparse_core_info()
    num_cores, num_subcores = info.num_cores, info.num_subcores
    num_workers = num_cores * num_subcores
    b_per_w = _BATCH // num_workers
    mesh = plsc.VectorSubcoreMesh(core_axis_name="c", subcore_axis_name="s")

    @functools.partial(
        pl.kernel,
        mesh=mesh,
        out_type=jax.ShapeDtypeStruct((_BATCH, _ACTIONS), jnp.float32),
        compiler_params=pltpu.CompilerParams(use_tc_tiling_on_sc=False),
        scratch_types=[
            pltpu.VMEM((b_per_w,), jnp.int32),
            pltpu.VMEM((b_per_w, _ACTIONS), jnp.float32),
            pltpu.SemaphoreType.DMA,
        ],
    )
    def gather_kernel(values_hbm, idx_hbm, out_hbm, idx_v, rows_v, sem):
        wid = lax.axis_index("s") * num_cores + lax.axis_index("c")
        base = wid * b_per_w
        pltpu.sync_copy(idx_hbm.at[pl.ds(base, b_per_w)], idx_v)
        pltpu.async_copy(values_hbm.at[idx_v], rows_v, sem).wait()
        pltpu.sync_copy(rows_v, out_hbm.at[pl.ds(base, b_per_w)])

    return gather_kernel


def kernel(state, values):
    idx = state.astype(jnp.int32)
    return _build_gather()(values, idx)


# speedup vs baseline: 17.1215x; 6.2586x over previous
import functools

import jax
import jax.numpy as jnp
from jax import lax
from jax.experimental import pallas as pl
from jax.experimental.pallas import tpu as pltpu
from jax.experimental.pallas import tpu_sc as plsc

_STATES = 1000000
_ACTIONS = 16
_BATCH = 16384
_DW = 76928
_NCHUNK = 13
_CHUNK_WORDS = _ACTIONS * _DW


def _detile_body(in_ref, out_ref):
    for j in range(_ACTIONS):
        out_ref[pl.ds(j * _DW, _DW)] = in_ref[0, j, :]


@functools.cache
def _build_detile():
    return pl.pallas_call(
        _detile_body,
        grid=(_NCHUNK,),
        in_specs=[pl.BlockSpec((1, _ACTIONS, _DW), lambda c: (0, 0, c))],
        out_specs=pl.BlockSpec((_CHUNK_WORDS,), lambda c: (c,)),
        out_shape=jax.ShapeDtypeStruct((_NCHUNK * _CHUNK_WORDS,), jnp.float32),
    )


@functools.cache
def _build_gather():
    info = plsc.get_sparse_core_info()
    num_cores, num_subcores = info.num_cores, info.num_subcores
    num_workers = num_cores * num_subcores
    b_per_w = _BATCH // num_workers
    n_blk = b_per_w // 16
    n_elems = b_per_w * _ACTIONS
    mesh = plsc.VectorSubcoreMesh(core_axis_name="c", subcore_axis_name="s")

    @functools.partial(
        pl.kernel,
        mesh=mesh,
        out_type=jax.ShapeDtypeStruct((_BATCH * _ACTIONS,), jnp.float32),
        compiler_params=pltpu.CompilerParams(needs_layout_passes=False),
        scratch_types=[
            pltpu.VMEM((b_per_w,), jnp.int32),
            pltpu.VMEM((n_elems,), jnp.int32),
            pltpu.VMEM((n_elems,), jnp.float32),
            pltpu.SemaphoreType.DMA,
        ],
    )
    def gather_kernel(flat_hbm, idx_hbm, out_hbm, idx_v, fidx_v, got_v, sem):
        wid = lax.axis_index("s") * num_cores + lax.axis_index("c")
        base = wid * b_per_w
        pltpu.sync_copy(idx_hbm.at[pl.ds(base, b_per_w)], idx_v)

        def fidx_body(k, _):
            v = idx_v[pl.ds(k * 16, 16)]
            ch = v // _DW
            word = ch * _CHUNK_WORDS + (v - ch * _DW)
            for j in range(_ACTIONS):
                fidx_v[pl.ds(j * b_per_w + k * 16, 16)] = word + j * _DW
            return _

        lax.fori_loop(0, n_blk, fidx_body, None)

        pltpu.async_copy(flat_hbm.at[fidx_v], got_v, sem).wait()

        for j in range(_ACTIONS):
            pltpu.sync_copy(
                got_v.at[pl.ds(j * b_per_w, b_per_w)],
                out_hbm.at[pl.ds(j * _BATCH + base, b_per_w)],
            )

    return gather_kernel


def kernel(state, values):
    idx = state.astype(jnp.int32)
    flat = _build_detile()(values.T.reshape(1, _ACTIONS, _STATES))
    out = _build_gather()(flat, idx)
    return out.reshape(_ACTIONS, _BATCH).T
